# baseline (speedup 1.0000x reference)
import functools

import jax
import jax.numpy as jnp
from jax import lax
from jax.experimental import pallas as pl
from jax.experimental.pallas import tpu as pltpu
from jax.experimental.pallas import tpu_sc as plsc

LANES = 16
N_SC = 2
N_SUB = 16
N_PASS = 2
VB = 782


def _tc_prep(base_t, bias_t):
    dim, vocab = base_t.shape
    vb_full = vocab // 128

    def body(b_ref, w_ref, o_ref):
        x = b_ref[...] + w_ref[...]
        for bb in range(vb_full):
            o_ref[0, bb, :, :] = x[:, bb * 128:(bb + 1) * 128]
        o_ref[0, vb_full, :, 0:vocab - vb_full * 128] = x[:, vb_full * 128:]

    return pl.pallas_call(
        body,
        grid=(dim // 8,),
        in_specs=[
            pl.BlockSpec((8, vocab), lambda a: (a, 0)),
            pl.BlockSpec((8, vocab), lambda a: (a, 0)),
        ],
        out_specs=pl.BlockSpec((1, VB, 8, 128), lambda a: (a, 0, 0, 0)),
        out_shape=jax.ShapeDtypeStruct((dim // 8, VB, 8, 128), jnp.float32),
    )(base_t, bias_t)


def _sc_embed_t(ids_t, w5):
    n_pos, n_batch = ids_t.shape
    n_a = w5.shape[0]
    dim = n_a * 8
    nc = n_batch // 128
    mesh = plsc.VectorSubcoreMesh(core_axis_name="c", subcore_axis_name="s")

    @functools.partial(
        pl.kernel,
        mesh=mesh,
        out_type=jax.ShapeDtypeStruct((n_pos, n_a, nc, 8 * 128), jnp.float32),
        scratch_types=[
            pltpu.VMEM((VB, 128), jnp.float32),
            pltpu.VMEM((n_batch,), jnp.int32),
            pltpu.VMEM((n_batch,), jnp.int32),
            pltpu.VMEM((nc, 128), jnp.float32),
            pltpu.VMEM((nc, 128), jnp.float32),
            pltpu.VMEM_SHARED((n_pos, n_batch), jnp.int32),
            pltpu.SemaphoreType.DMA,
            pltpu.SemaphoreType.DMA,
            pltpu.SemaphoreType.DMA,
            pltpu.SemaphoreType.DMA,
            pltpu.SemaphoreType.DMA,
            pltpu.SemaphoreType.DMA,
        ],
        compiler_params=pltpu.CompilerParams(
            use_tc_tiling_on_sc=False, needs_layout_passes=False),
    )
    def k(ids_hbm, w_hbm, out_hbm, lut, ids_a, ids_b, out_a, out_b,
          ids_sh, sem_d, sem_ia, sem_ib, sem_oa, sem_ob, sem_sh):
        c = lax.axis_index("c")
        s = lax.axis_index("s")
        zero16 = jnp.zeros((LANES,), jnp.int32)

        n_stage = (n_pos + N_SUB - 1) // N_SUB
        for t in range(n_stage):
            lr = N_SUB * t + s

            @pl.when(lr < n_pos)
            def _():
                pltpu.async_copy(ids_hbm.at[lr], ids_sh.at[lr], sem_sh)

        def start_ids(src, buf, sem, l):
            pltpu.async_copy(src.at[l], buf, sem)

        def wait_ids(src, buf, sem):
            pltpu.make_async_copy(src.at[0], buf, sem).wait()

        def out_dst(l, a, rm):
            return out_hbm.at[l, a, :, pl.ds(rm * 128, 128)]

        def gather_row(ids_buf, out_buf):
            def step(i, carry):
                for v in range(4):
                    for u in range(8):
                        sl = pl.ds((i * 4 + v) * 128 + u * LANES, LANES)
                        idx = ids_buf[sl]
                        out_buf[i * 4 + v, pl.ds(u * LANES, LANES)] = (
                            plsc.load_gather(lut, [zero16, idx]))
                return carry

            lax.fori_loop(0, nc // 4, step, 0)

        for p in range(N_PASS):
            src = ids_hbm if p == 0 else ids_sh
            d = 32 * c + 16 * p + s
            a = lax.div(d, 8)
            rm = lax.rem(d, 8)

            cp = pltpu.async_copy(
                w_hbm.at[a, :, pl.ds(rm * 128, 128)], lut, sem_d)

            if p == 1:
                for t in range(n_stage):
                    lr = N_SUB * t + s

                    @pl.when(lr < n_pos)
                    def _():
                        pltpu.make_async_copy(ids_hbm.at[0], ids_sh.at[0],
                                              sem_sh).wait()

                plsc.subcore_barrier()

            start_ids(src, ids_a, sem_ia, 0)
            cp.wait()

            def pos_pair(j, carry, src=src, a=a, rm=rm):
                la = 2 * j
                start_ids(src, ids_b, sem_ib, la + 1)
                wait_ids(src, ids_a, sem_ia)

                @pl.when(j > 0)
                def _():
                    pltpu.make_async_copy(out_a, out_dst(0, a, rm),
                                          sem_oa).wait()

                gather_row(ids_a, out_a)
                pltpu.async_copy(out_a, out_dst(la, a, rm), sem_oa)

                @pl.when(j < n_pos // 2 - 1)
                def _():
                    start_ids(src, ids_a, sem_ia, la + 2)

                wait_ids(src, ids_b, sem_ib)

                @pl.when(j > 0)
                def _():
                    pltpu.make_async_copy(out_b, out_dst(0, a, rm),
                                          sem_ob).wait()

                gather_row(ids_b, out_b)
                pltpu.async_copy(out_b, out_dst(la + 1, a, rm), sem_ob)
                return carry

            lax.fori_loop(0, n_pos // 2, pos_pair, 0)
            pltpu.make_async_copy(out_a, out_dst(0, a, rm), sem_oa).wait()
            pltpu.make_async_copy(out_b, out_dst(0, a, rm), sem_ob).wait()

    return k(ids_t, w5)


def kernel(input_ids, base_weight, bias):
    n_batch, n_pos = input_ids.shape
    dim = base_weight.shape[1]
    ids_t = input_ids.astype(jnp.int32).T
    w5 = _tc_prep(base_weight.T, bias.T)
    p = _sc_embed_t(ids_t, w5.reshape(8, VB, 1024))
    out = p.reshape(n_pos, 8, 32, 8, 128).transpose(2, 4, 0, 1, 3)
    return out.reshape(n_batch, n_pos, dim)


# speedup vs baseline: 1.0682x; 1.0682x over previous
import functools

import jax
import jax.numpy as jnp
from jax import lax
from jax.experimental import pallas as pl
from jax.experimental.pallas import tpu as pltpu
from jax.experimental.pallas import tpu_sc as plsc

LANES = 16
N_SC = 2
N_SUB = 16
N_PASS = 2
VB = 782


def _tc_prep(base_t, bias_t):
    dim, vocab = base_t.shape
    vb_full = vocab // 128

    def body(b_ref, w_ref, o_ref):
        x = b_ref[...] + w_ref[...]
        for bb in range(vb_full):
            o_ref[0, bb, :, :] = x[:, bb * 128:(bb + 1) * 128]
        o_ref[0, vb_full, :, 0:vocab - vb_full * 128] = x[:, vb_full * 128:]

    return pl.pallas_call(
        body,
        grid=(dim // 8,),
        in_specs=[
            pl.BlockSpec((8, vocab), lambda a: (a, 0)),
            pl.BlockSpec((8, vocab), lambda a: (a, 0)),
        ],
        out_specs=pl.BlockSpec((1, VB, 8, 128), lambda a: (a, 0, 0, 0)),
        out_shape=jax.ShapeDtypeStruct((dim // 8, VB, 8, 128), jnp.float32),
    )(base_t, bias_t)


def _sc_embed_t(ids_t, w5):
    n_pos, n_batch = ids_t.shape
    n_a = w5.shape[0]
    dim = n_a * 8
    nc = n_batch // 128
    mesh = plsc.VectorSubcoreMesh(core_axis_name="c", subcore_axis_name="s")

    @functools.partial(
        pl.kernel,
        mesh=mesh,
        out_type=jax.ShapeDtypeStruct((n_pos, n_a, nc, 8 * 128), jnp.float32),
        scratch_types=[
            pltpu.VMEM((VB, 128), jnp.float32),
            pltpu.VMEM((n_batch,), jnp.int32),
            pltpu.VMEM((n_batch,), jnp.int32),
            pltpu.VMEM((nc, 128), jnp.float32),
            pltpu.VMEM((nc, 128), jnp.float32),
            pltpu.VMEM_SHARED((n_pos, n_batch), jnp.int32),
            pltpu.SemaphoreType.DMA,
            pltpu.SemaphoreType.DMA,
            pltpu.SemaphoreType.DMA,
            pltpu.SemaphoreType.DMA,
            pltpu.SemaphoreType.DMA,
            pltpu.SemaphoreType.DMA,
        ],
        compiler_params=pltpu.CompilerParams(
            use_tc_tiling_on_sc=False, needs_layout_passes=False),
    )
    def k(ids_hbm, w_hbm, out_hbm, lut, ids_a, ids_b, out_a, out_b,
          ids_sh, sem_d, sem_ia, sem_ib, sem_oa, sem_ob, sem_sh):
        c = lax.axis_index("c")
        s = lax.axis_index("s")
        zero16 = jnp.zeros((LANES,), jnp.int32)

        n_stage = (n_pos + N_SUB - 1) // N_SUB
        for t in range(n_stage):
            lr = N_SUB * t + s

            @pl.when(lr < n_pos)
            def _():
                pltpu.async_copy(ids_hbm.at[lr], ids_sh.at[lr], sem_sh)

        def start_ids(src, buf, sem, l):
            pltpu.async_copy(src.at[l], buf, sem)

        def wait_ids(src, buf, sem):
            pltpu.make_async_copy(src.at[0], buf, sem).wait()

        def out_dst(l, a, rm):
            return out_hbm.at[l, a, :, pl.ds(rm * 128, 128)]

        def gather_row(ids_buf, out_buf):
            def step(i, carry):
                for v in range(4):
                    for u in range(8):
                        sl = pl.ds((i * 4 + v) * 128 + u * LANES, LANES)
                        idx = ids_buf[sl]
                        out_buf[i * 4 + v, pl.ds(u * LANES, LANES)] = (
                            plsc.load_gather(lut, [zero16, idx]))
                return carry

            lax.fori_loop(0, nc // 4, step, 0)

        for p in range(N_PASS):
            src = ids_sh
            d = 32 * c + 16 * p + s
            a = lax.div(d, 8)
            rm = lax.rem(d, 8)

            cp = pltpu.async_copy(
                w_hbm.at[a, :, pl.ds(rm * 128, 128)], lut, sem_d)

            if p == 0:
                for t in range(n_stage):
                    lr = N_SUB * t + s

                    @pl.when(lr < n_pos)
                    def _():
                        pltpu.make_async_copy(ids_hbm.at[0], ids_sh.at[0],
                                              sem_sh).wait()

                plsc.subcore_barrier()

            start_ids(src, ids_a, sem_ia, 0)
            cp.wait()

            def pos_pair(j, carry, src=src, a=a, rm=rm):
                la = 2 * j
                start_ids(src, ids_b, sem_ib, la + 1)
                wait_ids(src, ids_a, sem_ia)

                @pl.when(j > 0)
                def _():
                    pltpu.make_async_copy(out_a, out_dst(0, a, rm),
                                          sem_oa).wait()

                gather_row(ids_a, out_a)
                pltpu.async_copy(out_a, out_dst(la, a, rm), sem_oa)

                @pl.when(j < n_pos // 2 - 1)
                def _():
                    start_ids(src, ids_a, sem_ia, la + 2)

                wait_ids(src, ids_b, sem_ib)

                @pl.when(j > 0)
                def _():
                    pltpu.make_async_copy(out_b, out_dst(0, a, rm),
                                          sem_ob).wait()

                gather_row(ids_b, out_b)
                pltpu.async_copy(out_b, out_dst(la + 1, a, rm), sem_ob)
                return carry

            lax.fori_loop(0, n_pos // 2, pos_pair, 0)
            pltpu.make_async_copy(out_a, out_dst(0, a, rm), sem_oa).wait()
            pltpu.make_async_copy(out_b, out_dst(0, a, rm), sem_ob).wait()

    return k(ids_t, w5)


def kernel(input_ids, base_weight, bias):
    n_batch, n_pos = input_ids.shape
    dim = base_weight.shape[1]
    ids_t = input_ids.astype(jnp.int32).T
    w5 = _tc_prep(base_weight.T, bias.T)
    p = _sc_embed_t(ids_t, w5.reshape(8, VB, 1024))
    out = p.reshape(n_pos, 8, 32, 8, 128).transpose(2, 4, 0, 1, 3)
    return out.reshape(n_batch, n_pos, dim)
